# baseline (speedup 1.0000x reference)
import functools

import jax
import jax.numpy as jnp
from jax import lax
from jax.experimental import pallas as pl
from jax.experimental.pallas import tpu as pltpu
from jax.experimental.pallas import tpu_sc as plsc

N_NODES = 10000
FEAT = 128
E_EDGES = 320000

NCORES = 2
NSUB = 16
K = 128
CHUNKS_PER_SUB = 80
CHUNKS_PER_CORE = CHUNKS_PER_SUB * NSUB
EP = CHUNKS_PER_SUB * NCORES * NSUB * K

NP = 10240
ROWS_PER_SUB = NP // NSUB
ZROWS = 128
NZ = ROWS_PER_SUB // ZROWS

BLK = 2048
GRID = NP // BLK

_mesh = plsc.VectorSubcoreMesh(core_axis_name="c", subcore_axis_name="s")
_f32 = jnp.float32


@functools.partial(
    pl.kernel,
    out_type=jax.ShapeDtypeStruct((NCORES, NP, FEAT), _f32),
    mesh=_mesh,
    scratch_types=[
        pltpu.VMEM((K,), jnp.int32),
        pltpu.VMEM((K, FEAT), _f32),
        pltpu.VMEM_SHARED((NP, FEAT), _f32),
    ],
)
def _deg_kernel(dst_hbm, ones_hbm, zeros_hbm, out_hbm, didx, ov, acc):
    c = lax.axis_index("c")
    s = lax.axis_index("s")
    pltpu.sync_copy(zeros_hbm, ov.at[pl.ds(0, ZROWS)])

    @pl.loop(0, NZ)
    def _(b):
        pltpu.sync_copy(ov.at[pl.ds(0, ZROWS)],
                        acc.at[pl.ds(s * ROWS_PER_SUB + b * ZROWS, ZROWS)])

    pltpu.sync_copy(ones_hbm, ov)
    plsc.subcore_barrier()
    base = (c * CHUNKS_PER_CORE + s * CHUNKS_PER_SUB) * K

    @pl.loop(0, CHUNKS_PER_SUB)
    def _(j):
        pltpu.sync_copy(dst_hbm.at[pl.ds(base + j * K, K)], didx)
        pltpu.sync_copy(ov, acc.at[didx], add=True)

    plsc.subcore_barrier()

    @pl.loop(0, NZ)
    def _(b):
        r = s * ROWS_PER_SUB + b * ZROWS
        pltpu.sync_copy(acc.at[pl.ds(r, ZROWS)], out_hbm.at[c, pl.ds(r, ZROWS)])


@functools.partial(
    pl.kernel,
    out_type=jax.ShapeDtypeStruct((NCORES, NP, FEAT), _f32),
    mesh=_mesh,
    scratch_types=[
        pltpu.VMEM((K,), jnp.int32),
        pltpu.VMEM((K,), jnp.int32),
        pltpu.VMEM((K,), jnp.int32),
        pltpu.VMEM((K,), jnp.int32),
        pltpu.VMEM((K, FEAT), _f32),
        pltpu.VMEM((K, FEAT), _f32),
        pltpu.SemaphoreType.DMA,
        pltpu.SemaphoreType.DMA,
        pltpu.SemaphoreType.DMA,
        pltpu.SemaphoreType.DMA,
        pltpu.SemaphoreType.DMA,
        pltpu.SemaphoreType.DMA,
        pltpu.SemaphoreType.DMA,
        pltpu.SemaphoreType.DMA,
        pltpu.VMEM_SHARED((NP, FEAT), _f32),
    ],
)
def _edge_kernel(hs_hbm, src_hbm, dst_hbm, zeros_hbm, out_hbm,
                 sq0, sq1, dq0, dq1, rows0, rows1,
                 si0, si1, sd0, sd1, sg0, sg1, ss0, ss1, acc):
    c = lax.axis_index("c")
    s = lax.axis_index("s")
    pltpu.sync_copy(zeros_hbm, rows0.at[pl.ds(0, ZROWS)])

    @pl.loop(0, NZ)
    def _(b):
        pltpu.sync_copy(rows0.at[pl.ds(0, ZROWS)],
                        acc.at[pl.ds(s * ROWS_PER_SUB + b * ZROWS, ZROWS)])

    plsc.subcore_barrier()
    base = (c * CHUNKS_PER_CORE + s * CHUNKS_PER_SUB) * K
    slots = ((sq0, dq0, rows0, si0, sd0, sg0, ss0),
             (sq1, dq1, rows1, si1, sd1, sg1, ss1))

    for b in range(2):
        sq, _, _, si, _, _, _ = slots[b]
        pltpu.async_copy(src_hbm.at[pl.ds(base + b * K, K)], sq, si)

    for b in range(2):
        sq, dq, rows, si, sd, sg, ss = slots[b]
        pltpu.make_async_copy(src_hbm.at[pl.ds(base + b * K, K)], sq, si).wait()
        pltpu.async_copy(dst_hbm.at[pl.ds(base + b * K, K)], dq, sd)
        pltpu.async_copy(hs_hbm.at[sq], rows, sg)
        pltpu.make_async_copy(hs_hbm.at[sq], rows, sg).wait()
        pltpu.async_copy(src_hbm.at[pl.ds(base + (b + 2) * K, K)], sq, si)
        pltpu.make_async_copy(dst_hbm.at[pl.ds(base + b * K, K)], dq, sd).wait()
        pltpu.async_copy(rows, acc.at[dq], ss, add=True)

    @pl.loop(0, (CHUNKS_PER_SUB - 2) // 2)
    def _(jj):
        j = 2 + jj * 2
        for b in range(2):
            sq, dq, rows, si, sd, sg, ss = slots[b]
            e = base + (j + b) * K
            pltpu.make_async_copy(src_hbm.at[pl.ds(e, K)], sq, si).wait()
            pltpu.make_async_copy(rows, acc.at[dq], ss).wait()
            pltpu.async_copy(dst_hbm.at[pl.ds(e, K)], dq, sd)
            pltpu.async_copy(hs_hbm.at[sq], rows, sg)
            pltpu.make_async_copy(hs_hbm.at[sq], rows, sg).wait()
            pltpu.async_copy(src_hbm.at[pl.ds(e + 2 * K, K)], sq, si)
            pltpu.make_async_copy(dst_hbm.at[pl.ds(e, K)], dq, sd).wait()
            pltpu.async_copy(rows, acc.at[dq], ss, add=True)

    for b in range(2):
        sq, dq, rows, si, _, _, ss = slots[b]
        pltpu.make_async_copy(rows, acc.at[dq], ss).wait()
        pltpu.make_async_copy(src_hbm.at[pl.ds(base, K)], sq, si).wait()

    plsc.subcore_barrier()

    @pl.loop(0, NZ)
    def _(b):
        r = s * ROWS_PER_SUB + b * ZROWS
        pltpu.sync_copy(acc.at[pl.ds(r, ZROWS)], out_hbm.at[c, pl.ds(r, ZROWS)])


def _mm_body(x_ref, w_ref, o_ref):
    o_ref[...] = jnp.dot(x_ref[...], w_ref[...], preferred_element_type=_f32)


def _mm(x, w):
    return pl.pallas_call(
        _mm_body,
        grid=(GRID,),
        in_specs=[pl.BlockSpec((BLK, FEAT), lambda i: (i, 0)),
                  pl.BlockSpec((FEAT, FEAT), lambda i: (0, 0))],
        out_specs=pl.BlockSpec((BLK, FEAT), lambda i: (i, 0)),
        out_shape=jax.ShapeDtypeStruct((NP, FEAT), _f32),
    )(x, w)


def _scale_body(degp_ref, h_ref, hs_ref, dinv_ref):
    dinv = lax.rsqrt(degp_ref[0] + degp_ref[1] + 1.0)
    dinv_ref[...] = dinv
    hs_ref[...] = h_ref[...] * dinv


def _scale(degp, h):
    return pl.pallas_call(
        _scale_body,
        grid=(GRID,),
        in_specs=[pl.BlockSpec((NCORES, BLK, FEAT), lambda i: (0, i, 0)),
                  pl.BlockSpec((BLK, FEAT), lambda i: (i, 0))],
        out_specs=[pl.BlockSpec((BLK, FEAT), lambda i: (i, 0)),
                   pl.BlockSpec((BLK, FEAT), lambda i: (i, 0))],
        out_shape=[jax.ShapeDtypeStruct((NP, FEAT), _f32),
                   jax.ShapeDtypeStruct((NP, FEAT), _f32)],
    )(degp, h)


def _dense2_body(dinv_ref, accp_ref, h1_ref, b1_ref, w2_ref, h2_ref, hs2_ref):
    dinv = dinv_ref[...]
    ap = accp_ref[...]
    out1 = (ap[0] + ap[1]) * dinv + h1_ref[...] * dinv * dinv + b1_ref[...]
    h2 = jnp.dot(out1, w2_ref[...], preferred_element_type=_f32)
    h2_ref[...] = h2
    hs2_ref[...] = h2 * dinv


def _dense2(dinv, accp, h1, b1, w2):
    return pl.pallas_call(
        _dense2_body,
        grid=(GRID,),
        in_specs=[pl.BlockSpec((BLK, FEAT), lambda i: (i, 0)),
                  pl.BlockSpec((NCORES, BLK, FEAT), lambda i: (0, i, 0)),
                  pl.BlockSpec((BLK, FEAT), lambda i: (i, 0)),
                  pl.BlockSpec((1, FEAT), lambda i: (0, 0)),
                  pl.BlockSpec((FEAT, FEAT), lambda i: (0, 0))],
        out_specs=[pl.BlockSpec((BLK, FEAT), lambda i: (i, 0)),
                   pl.BlockSpec((BLK, FEAT), lambda i: (i, 0))],
        out_shape=[jax.ShapeDtypeStruct((NP, FEAT), _f32),
                   jax.ShapeDtypeStruct((NP, FEAT), _f32)],
    )(dinv, accp, h1, b1, w2)


def _final_body(dinv_ref, accp_ref, h2_ref, b2_ref, y_ref):
    dinv = dinv_ref[...]
    ap = accp_ref[...]
    out2 = (ap[0] + ap[1]) * dinv + h2_ref[...] * dinv * dinv + b2_ref[...]
    m = jnp.max(out2, axis=-1, keepdims=True)
    z = out2 - m
    y_ref[...] = z - jnp.log(jnp.sum(jnp.exp(z), axis=-1, keepdims=True))


def _final(dinv, accp, h2, b2):
    return pl.pallas_call(
        _final_body,
        grid=(GRID,),
        in_specs=[pl.BlockSpec((BLK, FEAT), lambda i: (i, 0)),
                  pl.BlockSpec((NCORES, BLK, FEAT), lambda i: (0, i, 0)),
                  pl.BlockSpec((BLK, FEAT), lambda i: (i, 0)),
                  pl.BlockSpec((1, FEAT), lambda i: (0, 0))],
        out_specs=pl.BlockSpec((BLK, FEAT), lambda i: (i, 0)),
        out_shape=jax.ShapeDtypeStruct((NP, FEAT), _f32),
    )(dinv, accp, h2, b2)


def kernel(x, edge_index, W1, b1, W2, b2):
    xp = jnp.zeros((NP, FEAT), _f32).at[:N_NODES].set(x)
    spad = jnp.full((EP + 2 * K - E_EDGES,), N_NODES, jnp.int32)
    dpad = jnp.full((EP - E_EDGES,), N_NODES, jnp.int32)
    src = jnp.concatenate([edge_index[0], spad])
    dst = jnp.concatenate([edge_index[1], dpad])
    ones128 = jnp.ones((K, FEAT), _f32)
    zeros128 = jnp.zeros((ZROWS, FEAT), _f32)

    degp = _deg_kernel(dst, ones128, zeros128)
    h1 = _mm(xp, W1)
    hs1, dinv = _scale(degp, h1)
    acc1 = _edge_kernel(hs1, src, dst, zeros128)
    h2, hs2 = _dense2(dinv, acc1, h1, b1.reshape(1, FEAT), W2)
    acc2 = _edge_kernel(hs2, src, dst, zeros128)
    y = _final(dinv, acc2, h2, b2.reshape(1, FEAT))
    return y[:N_NODES]


# speedup vs baseline: 1.0201x; 1.0201x over previous
import functools

import jax
import jax.numpy as jnp
from jax import lax
from jax.experimental import pallas as pl
from jax.experimental.pallas import tpu as pltpu
from jax.experimental.pallas import tpu_sc as plsc

N_NODES = 10000
FEAT = 128
E_EDGES = 320000

NCORES = 2
NSUB = 16
K = 128
CHUNKS_PER_SUB = 80
IB = 8
CHUNKS_PER_CORE = CHUNKS_PER_SUB * NSUB
EP = CHUNKS_PER_SUB * NCORES * NSUB * K

NP = 10240
ROWS_PER_SUB = NP // NSUB
ZROWS = 128
NZ = ROWS_PER_SUB // ZROWS

BLK = 2048
GRID = NP // BLK

_mesh = plsc.VectorSubcoreMesh(core_axis_name="c", subcore_axis_name="s")
_f32 = jnp.float32


@functools.partial(
    pl.kernel,
    out_type=jax.ShapeDtypeStruct((NCORES, NP, FEAT), _f32),
    mesh=_mesh,
    scratch_types=[
        pltpu.VMEM((IB, K), jnp.int32),
        pltpu.VMEM((K, FEAT), _f32),
        pltpu.VMEM_SHARED((NP, FEAT), _f32),
    ],
)
def _deg_kernel(dst_hbm, ones_hbm, zeros_hbm, out_hbm, dq, ov, acc):
    c = lax.axis_index("c")
    s = lax.axis_index("s")
    pltpu.sync_copy(zeros_hbm, ov.at[pl.ds(0, ZROWS)])

    @pl.loop(0, NZ)
    def _(b):
        pltpu.sync_copy(ov.at[pl.ds(0, ZROWS)],
                        acc.at[pl.ds(s * ROWS_PER_SUB + b * ZROWS, ZROWS)])

    pltpu.sync_copy(ones_hbm, ov)
    plsc.subcore_barrier()
    base = c * CHUNKS_PER_CORE + s * CHUNKS_PER_SUB

    @pl.loop(0, CHUNKS_PER_SUB // IB)
    def _(j):
        pltpu.sync_copy(dst_hbm.at[pl.ds(base + j * IB, IB)], dq)
        for q in range(IB):
            pltpu.sync_copy(ov, acc.at[dq.at[q]], add=True)

    plsc.subcore_barrier()

    @pl.loop(0, NZ)
    def _(b):
        r = s * ROWS_PER_SUB + b * ZROWS
        pltpu.sync_copy(acc.at[pl.ds(r, ZROWS)], out_hbm.at[c, pl.ds(r, ZROWS)])


@functools.partial(
    pl.kernel,
    out_type=jax.ShapeDtypeStruct((NCORES, NP, FEAT), _f32),
    mesh=_mesh,
    scratch_types=[
        pltpu.VMEM((IB, K), jnp.int32),
        pltpu.VMEM((IB, K), jnp.int32),
        pltpu.VMEM((K, FEAT), _f32),
        pltpu.VMEM_SHARED((NP, FEAT), _f32),
    ],
)
def _edge_kernel(hs_hbm, src_hbm, dst_hbm, zeros_hbm, out_hbm,
                 sq, dq, rows, acc):
    c = lax.axis_index("c")
    s = lax.axis_index("s")
    pltpu.sync_copy(zeros_hbm, rows.at[pl.ds(0, ZROWS)])

    @pl.loop(0, NZ)
    def _(b):
        pltpu.sync_copy(rows.at[pl.ds(0, ZROWS)],
                        acc.at[pl.ds(s * ROWS_PER_SUB + b * ZROWS, ZROWS)])

    plsc.subcore_barrier()
    base = c * CHUNKS_PER_CORE + s * CHUNKS_PER_SUB

    @pl.loop(0, CHUNKS_PER_SUB // IB)
    def _(j):
        e = base + j * IB
        pltpu.sync_copy(src_hbm.at[pl.ds(e, IB)], sq)
        pltpu.sync_copy(dst_hbm.at[pl.ds(e, IB)], dq)
        for q in range(IB):
            pltpu.sync_copy(hs_hbm.at[sq.at[q]], rows)
            pltpu.sync_copy(rows, acc.at[dq.at[q]], add=True)

    plsc.subcore_barrier()

    @pl.loop(0, NZ)
    def _(b):
        r = s * ROWS_PER_SUB + b * ZROWS
        pltpu.sync_copy(acc.at[pl.ds(r, ZROWS)], out_hbm.at[c, pl.ds(r, ZROWS)])


def _mm_body(x_ref, w_ref, o_ref):
    o_ref[...] = jnp.dot(x_ref[...], w_ref[...], preferred_element_type=_f32)


def _mm(x, w):
    return pl.pallas_call(
        _mm_body,
        grid=(GRID,),
        in_specs=[pl.BlockSpec((BLK, FEAT), lambda i: (i, 0)),
                  pl.BlockSpec((FEAT, FEAT), lambda i: (0, 0))],
        out_specs=pl.BlockSpec((BLK, FEAT), lambda i: (i, 0)),
        out_shape=jax.ShapeDtypeStruct((NP, FEAT), _f32),
    )(x, w)


def _scale_body(degp_ref, h_ref, hs_ref, dinv_ref):
    dinv = lax.rsqrt(degp_ref[0] + degp_ref[1] + 1.0)
    dinv_ref[...] = dinv
    hs_ref[...] = h_ref[...] * dinv


def _scale(degp, h):
    return pl.pallas_call(
        _scale_body,
        grid=(GRID,),
        in_specs=[pl.BlockSpec((NCORES, BLK, FEAT), lambda i: (0, i, 0)),
                  pl.BlockSpec((BLK, FEAT), lambda i: (i, 0))],
        out_specs=[pl.BlockSpec((BLK, FEAT), lambda i: (i, 0)),
                   pl.BlockSpec((BLK, FEAT), lambda i: (i, 0))],
        out_shape=[jax.ShapeDtypeStruct((NP, FEAT), _f32),
                   jax.ShapeDtypeStruct((NP, FEAT), _f32)],
    )(degp, h)


def _dense2_body(dinv_ref, accp_ref, h1_ref, b1_ref, w2_ref, h2_ref, hs2_ref):
    dinv = dinv_ref[...]
    ap = accp_ref[...]
    out1 = (ap[0] + ap[1]) * dinv + h1_ref[...] * dinv * dinv + b1_ref[...]
    h2 = jnp.dot(out1, w2_ref[...], preferred_element_type=_f32)
    h2_ref[...] = h2
    hs2_ref[...] = h2 * dinv


def _dense2(dinv, accp, h1, b1, w2):
    return pl.pallas_call(
        _dense2_body,
        grid=(GRID,),
        in_specs=[pl.BlockSpec((BLK, FEAT), lambda i: (i, 0)),
                  pl.BlockSpec((NCORES, BLK, FEAT), lambda i: (0, i, 0)),
                  pl.BlockSpec((BLK, FEAT), lambda i: (i, 0)),
                  pl.BlockSpec((1, FEAT), lambda i: (0, 0)),
                  pl.BlockSpec((FEAT, FEAT), lambda i: (0, 0))],
        out_specs=[pl.BlockSpec((BLK, FEAT), lambda i: (i, 0)),
                   pl.BlockSpec((BLK, FEAT), lambda i: (i, 0))],
        out_shape=[jax.ShapeDtypeStruct((NP, FEAT), _f32),
                   jax.ShapeDtypeStruct((NP, FEAT), _f32)],
    )(dinv, accp, h1, b1, w2)


def _final_body(dinv_ref, accp_ref, h2_ref, b2_ref, y_ref):
    dinv = dinv_ref[...]
    ap = accp_ref[...]
    out2 = (ap[0] + ap[1]) * dinv + h2_ref[...] * dinv * dinv + b2_ref[...]
    m = jnp.max(out2, axis=-1, keepdims=True)
    z = out2 - m
    y_ref[...] = z - jnp.log(jnp.sum(jnp.exp(z), axis=-1, keepdims=True))


def _final(dinv, accp, h2, b2):
    return pl.pallas_call(
        _final_body,
        grid=(GRID,),
        in_specs=[pl.BlockSpec((BLK, FEAT), lambda i: (i, 0)),
                  pl.BlockSpec((NCORES, BLK, FEAT), lambda i: (0, i, 0)),
                  pl.BlockSpec((BLK, FEAT), lambda i: (i, 0)),
                  pl.BlockSpec((1, FEAT), lambda i: (0, 0))],
        out_specs=pl.BlockSpec((BLK, FEAT), lambda i: (i, 0)),
        out_shape=jax.ShapeDtypeStruct((NP, FEAT), _f32),
    )(dinv, accp, h2, b2)


def kernel(x, edge_index, W1, b1, W2, b2):
    xp = jnp.zeros((NP, FEAT), _f32).at[:N_NODES].set(x)
    pad = jnp.full((EP - E_EDGES,), N_NODES, jnp.int32)
    src = jnp.concatenate([edge_index[0], pad]).reshape(EP // K, K)
    dst = jnp.concatenate([edge_index[1], pad]).reshape(EP // K, K)
    ones128 = jnp.ones((K, FEAT), _f32)
    zeros128 = jnp.zeros((ZROWS, FEAT), _f32)

    degp = _deg_kernel(dst, ones128, zeros128)
    h1 = _mm(xp, W1)
    hs1, dinv = _scale(degp, h1)
    acc1 = _edge_kernel(hs1, src, dst, zeros128)
    h2, hs2 = _dense2(dinv, acc1, h1, b1.reshape(1, FEAT), W2)
    acc2 = _edge_kernel(hs2, src, dst, zeros128)
    y = _final(dinv, acc2, h2, b2.reshape(1, FEAT))
    return y[:N_NODES]


# speedup vs baseline: 1.2237x; 1.1996x over previous
import functools

import jax
import jax.numpy as jnp
from jax import lax
from jax.experimental import pallas as pl
from jax.experimental.pallas import tpu as pltpu
from jax.experimental.pallas import tpu_sc as plsc

N_NODES = 10000
FEAT = 128
E_EDGES = 320000

NCORES = 2
NSUB = 16
K = 128
CHUNKS_PER_SUB = 79
CHUNKS_PER_CORE = CHUNKS_PER_SUB * NSUB
EP = CHUNKS_PER_SUB * NCORES * NSUB * K

NP = 10240
ROWS_PER_SUB = NP // NSUB
ZROWS = 128
NZ = ROWS_PER_SUB // ZROWS

BLK = 2048
GRID = NP // BLK

_mesh = plsc.VectorSubcoreMesh(core_axis_name="c", subcore_axis_name="s")
_f32 = jnp.float32


@functools.partial(
    pl.kernel,
    out_type=jax.ShapeDtypeStruct((NCORES, NP, FEAT), _f32),
    mesh=_mesh,
    scratch_types=[
        pltpu.VMEM((K,), jnp.int32),
        pltpu.VMEM((K, FEAT), _f32),
        pltpu.VMEM_SHARED((NP, FEAT), _f32),
    ],
)
def _deg_kernel(dst_hbm, ones_hbm, zeros_hbm, out_hbm, didx, ov, acc):
    c = lax.axis_index("c")
    s = lax.axis_index("s")
    pltpu.sync_copy(zeros_hbm, ov.at[pl.ds(0, ZROWS)])

    @pl.loop(0, NZ)
    def _(b):
        pltpu.sync_copy(ov.at[pl.ds(0, ZROWS)],
                        acc.at[pl.ds(s * ROWS_PER_SUB + b * ZROWS, ZROWS)])

    pltpu.sync_copy(ones_hbm, ov)
    plsc.subcore_barrier()
    base = (c * CHUNKS_PER_CORE + s * CHUNKS_PER_SUB) * K

    @pl.loop(0, CHUNKS_PER_SUB)
    def _(j):
        pltpu.sync_copy(dst_hbm.at[pl.ds(base + j * K, K)], didx)
        pltpu.sync_copy(ov, acc.at[didx], add=True)

    plsc.subcore_barrier()

    @pl.loop(0, NZ)
    def _(b):
        r = s * ROWS_PER_SUB + b * ZROWS
        pltpu.sync_copy(acc.at[pl.ds(r, ZROWS)], out_hbm.at[c, pl.ds(r, ZROWS)])


@functools.partial(
    pl.kernel,
    out_type=jax.ShapeDtypeStruct((NCORES, NP, FEAT), _f32),
    mesh=_mesh,
    scratch_types=[
        pltpu.VMEM((K,), jnp.int32),
        pltpu.VMEM((K,), jnp.int32),
        pltpu.VMEM((K, FEAT), _f32),
        pltpu.VMEM_SHARED((NP, FEAT), _f32),
    ],
)
def _edge_kernel(hs_hbm, src_hbm, dst_hbm, zeros_hbm, out_hbm,
                 sidx, didx, rows, acc):
    c = lax.axis_index("c")
    s = lax.axis_index("s")
    pltpu.sync_copy(zeros_hbm, rows.at[pl.ds(0, ZROWS)])

    @pl.loop(0, NZ)
    def _(b):
        pltpu.sync_copy(rows.at[pl.ds(0, ZROWS)],
                        acc.at[pl.ds(s * ROWS_PER_SUB + b * ZROWS, ZROWS)])

    plsc.subcore_barrier()
    base = (c * CHUNKS_PER_CORE + s * CHUNKS_PER_SUB) * K

    @pl.loop(0, CHUNKS_PER_SUB)
    def _(j):
        e = base + j * K
        pltpu.sync_copy(src_hbm.at[pl.ds(e, K)], sidx)
        pltpu.sync_copy(dst_hbm.at[pl.ds(e, K)], didx)
        pltpu.sync_copy(hs_hbm.at[sidx], rows)
        pltpu.sync_copy(rows, acc.at[didx], add=True)

    plsc.subcore_barrier()

    @pl.loop(0, NZ)
    def _(b):
        r = s * ROWS_PER_SUB + b * ZROWS
        pltpu.sync_copy(acc.at[pl.ds(r, ZROWS)], out_hbm.at[c, pl.ds(r, ZROWS)])


def _mm_body(x_ref, w_ref, o_ref):
    o_ref[...] = jnp.dot(x_ref[...], w_ref[...], preferred_element_type=_f32)


def _mm(x, w):
    return pl.pallas_call(
        _mm_body,
        grid=(GRID,),
        in_specs=[pl.BlockSpec((BLK, FEAT), lambda i: (i, 0)),
                  pl.BlockSpec((FEAT, FEAT), lambda i: (0, 0))],
        out_specs=pl.BlockSpec((BLK, FEAT), lambda i: (i, 0)),
        out_shape=jax.ShapeDtypeStruct((NP, FEAT), _f32),
    )(x, w)


def _scale_body(degp_ref, h_ref, hs_ref, dinv_ref):
    dinv = lax.rsqrt(degp_ref[0] + degp_ref[1] + 1.0)
    dinv_ref[...] = dinv
    hs_ref[...] = h_ref[...] * dinv


def _scale(degp, h):
    return pl.pallas_call(
        _scale_body,
        grid=(GRID,),
        in_specs=[pl.BlockSpec((NCORES, BLK, FEAT), lambda i: (0, i, 0)),
                  pl.BlockSpec((BLK, FEAT), lambda i: (i, 0))],
        out_specs=[pl.BlockSpec((BLK, FEAT), lambda i: (i, 0)),
                   pl.BlockSpec((BLK, FEAT), lambda i: (i, 0))],
        out_shape=[jax.ShapeDtypeStruct((NP, FEAT), _f32),
                   jax.ShapeDtypeStruct((NP, FEAT), _f32)],
    )(degp, h)


def _dense2_body(dinv_ref, accp_ref, h1_ref, b1_ref, w2_ref, h2_ref, hs2_ref):
    dinv = dinv_ref[...]
    ap = accp_ref[...]
    out1 = (ap[0] + ap[1]) * dinv + h1_ref[...] * dinv * dinv + b1_ref[...]
    h2 = jnp.dot(out1, w2_ref[...], preferred_element_type=_f32)
    h2_ref[...] = h2
    hs2_ref[...] = h2 * dinv


def _dense2(dinv, accp, h1, b1, w2):
    return pl.pallas_call(
        _dense2_body,
        grid=(GRID,),
        in_specs=[pl.BlockSpec((BLK, FEAT), lambda i: (i, 0)),
                  pl.BlockSpec((NCORES, BLK, FEAT), lambda i: (0, i, 0)),
                  pl.BlockSpec((BLK, FEAT), lambda i: (i, 0)),
                  pl.BlockSpec((1, FEAT), lambda i: (0, 0)),
                  pl.BlockSpec((FEAT, FEAT), lambda i: (0, 0))],
        out_specs=[pl.BlockSpec((BLK, FEAT), lambda i: (i, 0)),
                   pl.BlockSpec((BLK, FEAT), lambda i: (i, 0))],
        out_shape=[jax.ShapeDtypeStruct((NP, FEAT), _f32),
                   jax.ShapeDtypeStruct((NP, FEAT), _f32)],
    )(dinv, accp, h1, b1, w2)


def _final_body(dinv_ref, accp_ref, h2_ref, b2_ref, y_ref):
    dinv = dinv_ref[...]
    ap = accp_ref[...]
    out2 = (ap[0] + ap[1]) * dinv + h2_ref[...] * dinv * dinv + b2_ref[...]
    m = jnp.max(out2, axis=-1, keepdims=True)
    z = out2 - m
    y_ref[...] = z - jnp.log(jnp.sum(jnp.exp(z), axis=-1, keepdims=True))


def _final(dinv, accp, h2, b2):
    return pl.pallas_call(
        _final_body,
        grid=(GRID,),
        in_specs=[pl.BlockSpec((BLK, FEAT), lambda i: (i, 0)),
                  pl.BlockSpec((NCORES, BLK, FEAT), lambda i: (0, i, 0)),
                  pl.BlockSpec((BLK, FEAT), lambda i: (i, 0)),
                  pl.BlockSpec((1, FEAT), lambda i: (0, 0))],
        out_specs=pl.BlockSpec((BLK, FEAT), lambda i: (i, 0)),
        out_shape=jax.ShapeDtypeStruct((NP, FEAT), _f32),
    )(dinv, accp, h2, b2)


def kernel(x, edge_index, W1, b1, W2, b2):
    xp = jnp.zeros((NP, FEAT), _f32).at[:N_NODES].set(x)
    pad = jnp.full((EP - E_EDGES,), N_NODES, jnp.int32)
    src = jnp.concatenate([edge_index[0], pad])
    dst = jnp.concatenate([edge_index[1], pad])
    ones128 = jnp.ones((K, FEAT), _f32)
    zeros128 = jnp.zeros((ZROWS, FEAT), _f32)

    degp = _deg_kernel(dst, ones128, zeros128)
    h1 = _mm(xp, W1)
    hs1, dinv = _scale(degp, h1)
    acc1 = _edge_kernel(hs1, src, dst, zeros128)
    h2, hs2 = _dense2(dinv, acc1, h1, b1.reshape(1, FEAT), W2)
    acc2 = _edge_kernel(hs2, src, dst, zeros128)
    y = _final(dinv, acc2, h2, b2.reshape(1, FEAT))
    return y[:N_NODES]


# speedup vs baseline: 1.4082x; 1.1508x over previous
import functools

import jax
import jax.numpy as jnp
from jax import lax
from jax.experimental import pallas as pl
from jax.experimental.pallas import tpu as pltpu
from jax.experimental.pallas import tpu_sc as plsc

N_NODES = 10000
FEAT = 128
E_EDGES = 320000

NCORES = 2
NSUB = 16
K = 128
CHUNKS_PER_SUB = 79
CHUNKS_PER_CORE = CHUNKS_PER_SUB * NSUB
EP = CHUNKS_PER_SUB * NCORES * NSUB * K

NP = 10240
ROWS_PER_SUB = NP // NSUB
ZROWS = 128
NZ = ROWS_PER_SUB // ZROWS

BLK = 2048
GRID = NP // BLK

_mesh = plsc.VectorSubcoreMesh(core_axis_name="c", subcore_axis_name="s")
_f32 = jnp.float32


@functools.partial(
    pl.kernel,
    out_type=jax.ShapeDtypeStruct((NCORES, NP, FEAT), _f32),
    mesh=_mesh,
    scratch_types=[
        pltpu.VMEM((K,), jnp.int32),
        pltpu.VMEM((K,), jnp.int32),
        pltpu.VMEM((K, FEAT), _f32),
        pltpu.SemaphoreType.DMA,
        pltpu.SemaphoreType.DMA,
        pltpu.VMEM_SHARED((NP, FEAT), _f32),
    ],
)
def _deg_kernel(dst_hbm, ones_hbm, zeros_hbm, out_hbm,
                didx0, didx1, ov, ss0, ss1, acc):
    c = lax.axis_index("c")
    s = lax.axis_index("s")
    pltpu.sync_copy(zeros_hbm, ov.at[pl.ds(0, ZROWS)])

    @pl.loop(0, NZ)
    def _(b):
        pltpu.sync_copy(ov.at[pl.ds(0, ZROWS)],
                        acc.at[pl.ds(s * ROWS_PER_SUB + b * ZROWS, ZROWS)])

    pltpu.sync_copy(ones_hbm, ov)
    plsc.subcore_barrier()
    base = (c * CHUNKS_PER_CORE + s * CHUNKS_PER_SUB) * K
    slots = ((didx0, ss0), (didx1, ss1))

    for b in range(2):
        didx, ss = slots[b]
        pltpu.sync_copy(dst_hbm.at[pl.ds(base + b * K, K)], didx)
        pltpu.async_copy(ov, acc.at[didx], ss, add=True)

    @pl.loop(0, (CHUNKS_PER_SUB - 2) // 2)
    def _(jj):
        j = 2 + jj * 2
        for b in range(2):
            didx, ss = slots[b]
            pltpu.make_async_copy(ov, acc.at[didx], ss).wait()
            pltpu.sync_copy(dst_hbm.at[pl.ds(base + (j + b) * K, K)], didx)
            pltpu.async_copy(ov, acc.at[didx], ss, add=True)

    pltpu.make_async_copy(ov, acc.at[didx0], ss0).wait()
    pltpu.sync_copy(dst_hbm.at[pl.ds(base + (CHUNKS_PER_SUB - 1) * K, K)],
                    didx0)
    pltpu.async_copy(ov, acc.at[didx0], ss0, add=True)
    pltpu.make_async_copy(ov, acc.at[didx0], ss0).wait()
    pltpu.make_async_copy(ov, acc.at[didx1], ss1).wait()

    plsc.subcore_barrier()

    @pl.loop(0, NZ)
    def _(b):
        r = s * ROWS_PER_SUB + b * ZROWS
        pltpu.sync_copy(acc.at[pl.ds(r, ZROWS)], out_hbm.at[c, pl.ds(r, ZROWS)])


@functools.partial(
    pl.kernel,
    out_type=jax.ShapeDtypeStruct((NCORES, NP, FEAT), _f32),
    mesh=_mesh,
    scratch_types=[
        pltpu.VMEM((K,), jnp.int32),
        pltpu.VMEM((K,), jnp.int32),
        pltpu.VMEM((K,), jnp.int32),
        pltpu.VMEM((K, FEAT), _f32),
        pltpu.VMEM((K, FEAT), _f32),
        pltpu.SemaphoreType.DMA,
        pltpu.SemaphoreType.DMA,
        pltpu.VMEM_SHARED((NP, FEAT), _f32),
    ],
)
def _edge_kernel(hs_hbm, src_hbm, dst_hbm, zeros_hbm, out_hbm,
                 sidx, didx0, didx1, rows0, rows1, ss0, ss1, acc):
    c = lax.axis_index("c")
    s = lax.axis_index("s")
    pltpu.sync_copy(zeros_hbm, rows0.at[pl.ds(0, ZROWS)])

    @pl.loop(0, NZ)
    def _(b):
        pltpu.sync_copy(rows0.at[pl.ds(0, ZROWS)],
                        acc.at[pl.ds(s * ROWS_PER_SUB + b * ZROWS, ZROWS)])

    plsc.subcore_barrier()
    base = (c * CHUNKS_PER_CORE + s * CHUNKS_PER_SUB) * K
    slots = ((didx0, rows0, ss0), (didx1, rows1, ss1))

    def _chunk(j, b, first):
        didx, rows, ss = slots[b]
        e = base + j * K
        if not first:
            pltpu.make_async_copy(rows, acc.at[didx], ss).wait()
        pltpu.sync_copy(src_hbm.at[pl.ds(e, K)], sidx)
        pltpu.sync_copy(dst_hbm.at[pl.ds(e, K)], didx)
        pltpu.sync_copy(hs_hbm.at[sidx], rows)
        pltpu.async_copy(rows, acc.at[didx], ss, add=True)

    for b in range(2):
        _chunk(b, b, True)

    @pl.loop(0, (CHUNKS_PER_SUB - 2) // 2)
    def _(jj):
        for b in range(2):
            _chunk(2 + jj * 2 + b, b, False)

    _chunk(CHUNKS_PER_SUB - 1, 0, False)
    pltpu.make_async_copy(rows0, acc.at[didx0], ss0).wait()
    pltpu.make_async_copy(rows1, acc.at[didx1], ss1).wait()

    plsc.subcore_barrier()

    @pl.loop(0, NZ)
    def _(b):
        r = s * ROWS_PER_SUB + b * ZROWS
        pltpu.sync_copy(acc.at[pl.ds(r, ZROWS)], out_hbm.at[c, pl.ds(r, ZROWS)])


def _mm_body(x_ref, w_ref, o_ref):
    o_ref[...] = jnp.dot(x_ref[...], w_ref[...], preferred_element_type=_f32)


def _mm(x, w):
    return pl.pallas_call(
        _mm_body,
        grid=(GRID,),
        in_specs=[pl.BlockSpec((BLK, FEAT), lambda i: (i, 0)),
                  pl.BlockSpec((FEAT, FEAT), lambda i: (0, 0))],
        out_specs=pl.BlockSpec((BLK, FEAT), lambda i: (i, 0)),
        out_shape=jax.ShapeDtypeStruct((NP, FEAT), _f32),
    )(x, w)


def _scale_body(degp_ref, h_ref, hs_ref, dinv_ref):
    dinv = lax.rsqrt(degp_ref[0] + degp_ref[1] + 1.0)
    dinv_ref[...] = dinv
    hs_ref[...] = h_ref[...] * dinv


def _scale(degp, h):
    return pl.pallas_call(
        _scale_body,
        grid=(GRID,),
        in_specs=[pl.BlockSpec((NCORES, BLK, FEAT), lambda i: (0, i, 0)),
                  pl.BlockSpec((BLK, FEAT), lambda i: (i, 0))],
        out_specs=[pl.BlockSpec((BLK, FEAT), lambda i: (i, 0)),
                   pl.BlockSpec((BLK, FEAT), lambda i: (i, 0))],
        out_shape=[jax.ShapeDtypeStruct((NP, FEAT), _f32),
                   jax.ShapeDtypeStruct((NP, FEAT), _f32)],
    )(degp, h)


def _dense2_body(dinv_ref, accp_ref, h1_ref, b1_ref, w2_ref, h2_ref, hs2_ref):
    dinv = dinv_ref[...]
    ap = accp_ref[...]
    out1 = (ap[0] + ap[1]) * dinv + h1_ref[...] * dinv * dinv + b1_ref[...]
    h2 = jnp.dot(out1, w2_ref[...], preferred_element_type=_f32)
    h2_ref[...] = h2
    hs2_ref[...] = h2 * dinv


def _dense2(dinv, accp, h1, b1, w2):
    return pl.pallas_call(
        _dense2_body,
        grid=(GRID,),
        in_specs=[pl.BlockSpec((BLK, FEAT), lambda i: (i, 0)),
                  pl.BlockSpec((NCORES, BLK, FEAT), lambda i: (0, i, 0)),
                  pl.BlockSpec((BLK, FEAT), lambda i: (i, 0)),
                  pl.BlockSpec((1, FEAT), lambda i: (0, 0)),
                  pl.BlockSpec((FEAT, FEAT), lambda i: (0, 0))],
        out_specs=[pl.BlockSpec((BLK, FEAT), lambda i: (i, 0)),
                   pl.BlockSpec((BLK, FEAT), lambda i: (i, 0))],
        out_shape=[jax.ShapeDtypeStruct((NP, FEAT), _f32),
                   jax.ShapeDtypeStruct((NP, FEAT), _f32)],
    )(dinv, accp, h1, b1, w2)


def _final_body(dinv_ref, accp_ref, h2_ref, b2_ref, y_ref):
    dinv = dinv_ref[...]
    ap = accp_ref[...]
    out2 = (ap[0] + ap[1]) * dinv + h2_ref[...] * dinv * dinv + b2_ref[...]
    m = jnp.max(out2, axis=-1, keepdims=True)
    z = out2 - m
    y_ref[...] = z - jnp.log(jnp.sum(jnp.exp(z), axis=-1, keepdims=True))


def _final(dinv, accp, h2, b2):
    return pl.pallas_call(
        _final_body,
        grid=(GRID,),
        in_specs=[pl.BlockSpec((BLK, FEAT), lambda i: (i, 0)),
                  pl.BlockSpec((NCORES, BLK, FEAT), lambda i: (0, i, 0)),
                  pl.BlockSpec((BLK, FEAT), lambda i: (i, 0)),
                  pl.BlockSpec((1, FEAT), lambda i: (0, 0))],
        out_specs=pl.BlockSpec((BLK, FEAT), lambda i: (i, 0)),
        out_shape=jax.ShapeDtypeStruct((NP, FEAT), _f32),
    )(dinv, accp, h2, b2)


def kernel(x, edge_index, W1, b1, W2, b2):
    xp = jnp.zeros((NP, FEAT), _f32).at[:N_NODES].set(x)
    pad = jnp.full((EP - E_EDGES,), N_NODES, jnp.int32)
    src = jnp.concatenate([edge_index[0], pad])
    dst = jnp.concatenate([edge_index[1], pad])
    ones128 = jnp.ones((K, FEAT), _f32)
    zeros128 = jnp.zeros((ZROWS, FEAT), _f32)

    degp = _deg_kernel(dst, ones128, zeros128)
    h1 = _mm(xp, W1)
    hs1, dinv = _scale(degp, h1)
    acc1 = _edge_kernel(hs1, src, dst, zeros128)
    h2, hs2 = _dense2(dinv, acc1, h1, b1.reshape(1, FEAT), W2)
    acc2 = _edge_kernel(hs2, src, dst, zeros128)
    y = _final(dinv, acc2, h2, b2.reshape(1, FEAT))
    return y[:N_NODES]


# speedup vs baseline: 1.6383x; 1.1634x over previous
import functools

import jax
import jax.numpy as jnp
from jax import lax
from jax.experimental import pallas as pl
from jax.experimental.pallas import tpu as pltpu
from jax.experimental.pallas import tpu_sc as plsc

N_NODES = 10000
FEAT = 128
E_EDGES = 320000

NCORES = 2
NSUB = 16
K = 128
CHUNKS_PER_SUB = 79
CHUNKS_PER_CORE = CHUNKS_PER_SUB * NSUB
EP = CHUNKS_PER_SUB * NCORES * NSUB * K

NP = 10240
ROWS_PER_SUB = NP // NSUB
ZROWS = 128
NZ = ROWS_PER_SUB // ZROWS

BLK = 2048
GRID = NP // BLK

_mesh = plsc.VectorSubcoreMesh(core_axis_name="c", subcore_axis_name="s")
_f32 = jnp.float32


@functools.partial(
    pl.kernel,
    out_type=jax.ShapeDtypeStruct((NCORES, NP, FEAT), _f32),
    mesh=_mesh,
    scratch_types=[
        pltpu.VMEM((K,), jnp.int32),
        pltpu.VMEM((K,), jnp.int32),
        pltpu.VMEM((K, FEAT), _f32),
        pltpu.SemaphoreType.DMA,
        pltpu.SemaphoreType.DMA,
        pltpu.VMEM_SHARED((NP, FEAT), _f32),
    ],
)
def _deg_kernel(dst_hbm, ones_hbm, zeros_hbm, out_hbm,
                didx0, didx1, ov, ss0, ss1, acc):
    c = lax.axis_index("c")
    s = lax.axis_index("s")
    pltpu.sync_copy(zeros_hbm, ov.at[pl.ds(0, ZROWS)])

    @pl.loop(0, NZ)
    def _(b):
        pltpu.sync_copy(ov.at[pl.ds(0, ZROWS)],
                        acc.at[pl.ds(s * ROWS_PER_SUB + b * ZROWS, ZROWS)])

    pltpu.sync_copy(ones_hbm, ov)
    plsc.subcore_barrier()
    base = (c * CHUNKS_PER_CORE + s * CHUNKS_PER_SUB) * K
    slots = ((didx0, ss0), (didx1, ss1))

    for b in range(2):
        didx, ss = slots[b]
        pltpu.sync_copy(dst_hbm.at[pl.ds(base + b * K, K)], didx)
        pltpu.async_copy(ov, acc.at[didx], ss, add=True)

    @pl.loop(0, (CHUNKS_PER_SUB - 2) // 2)
    def _(jj):
        j = 2 + jj * 2
        for b in range(2):
            didx, ss = slots[b]
            pltpu.make_async_copy(ov, acc.at[didx], ss).wait()
            pltpu.sync_copy(dst_hbm.at[pl.ds(base + (j + b) * K, K)], didx)
            pltpu.async_copy(ov, acc.at[didx], ss, add=True)

    pltpu.make_async_copy(ov, acc.at[didx0], ss0).wait()
    pltpu.sync_copy(dst_hbm.at[pl.ds(base + (CHUNKS_PER_SUB - 1) * K, K)],
                    didx0)
    pltpu.async_copy(ov, acc.at[didx0], ss0, add=True)
    pltpu.make_async_copy(ov, acc.at[didx0], ss0).wait()
    pltpu.make_async_copy(ov, acc.at[didx1], ss1).wait()

    plsc.subcore_barrier()

    @pl.loop(0, NZ)
    def _(b):
        r = s * ROWS_PER_SUB + b * ZROWS
        pltpu.sync_copy(acc.at[pl.ds(r, ZROWS)], out_hbm.at[c, pl.ds(r, ZROWS)])


@functools.partial(
    pl.kernel,
    out_type=jax.ShapeDtypeStruct((NCORES, NP, FEAT), _f32),
    mesh=_mesh,
    scratch_types=[
        pltpu.VMEM((K,), jnp.int32),
        pltpu.VMEM((K,), jnp.int32),
        pltpu.VMEM((K,), jnp.int32),
        pltpu.VMEM((K,), jnp.int32),
        pltpu.VMEM((K, FEAT), _f32),
        pltpu.VMEM((K, FEAT), _f32),
        pltpu.SemaphoreType.DMA,
        pltpu.SemaphoreType.DMA,
        pltpu.SemaphoreType.DMA,
        pltpu.SemaphoreType.DMA,
        pltpu.SemaphoreType.DMA,
        pltpu.SemaphoreType.DMA,
        pltpu.SemaphoreType.DMA,
        pltpu.SemaphoreType.DMA,
        pltpu.VMEM_SHARED((NP, FEAT), _f32),
    ],
)
def _edge_kernel(hs_hbm, src_hbm, dst_hbm, zeros_hbm, out_hbm,
                 sidx0, sidx1, didx0, didx1, rows0, rows1,
                 si0, si1, sd0, sd1, sg0, sg1, ss0, ss1, acc):
    c = lax.axis_index("c")
    s = lax.axis_index("s")
    pltpu.sync_copy(zeros_hbm, rows0.at[pl.ds(0, ZROWS)])

    @pl.loop(0, NZ)
    def _(b):
        pltpu.sync_copy(rows0.at[pl.ds(0, ZROWS)],
                        acc.at[pl.ds(s * ROWS_PER_SUB + b * ZROWS, ZROWS)])

    plsc.subcore_barrier()
    base = (c * CHUNKS_PER_CORE + s * CHUNKS_PER_SUB) * K
    slots = ((sidx0, didx0, rows0, si0, sd0, sg0, ss0),
             (sidx1, didx1, rows1, si1, sd1, sg1, ss1))

    def _body(j, b, prefetch):
        sidx, didx, rows, si, sd, sg, ss = slots[b]
        osidx, odidx, orows, osi, osd, _, oss = slots[1 - b]
        e = base + j * K
        pltpu.make_async_copy(src_hbm.at[pl.ds(e, K)], sidx, si).wait()
        pltpu.make_async_copy(dst_hbm.at[pl.ds(e, K)], didx, sd).wait()
        pltpu.async_copy(hs_hbm.at[sidx], rows, sg)
        if prefetch:
            ne = base + (j + 1) * K
            pltpu.make_async_copy(orows, acc.at[odidx], oss).wait()
            pltpu.async_copy(src_hbm.at[pl.ds(ne, K)], osidx, osi)
            pltpu.async_copy(dst_hbm.at[pl.ds(ne, K)], odidx, osd)
        pltpu.make_async_copy(hs_hbm.at[sidx], rows, sg).wait()
        pltpu.async_copy(rows, acc.at[didx], ss, add=True)

    pltpu.sync_copy(src_hbm.at[pl.ds(base, K)], sidx0)
    pltpu.sync_copy(dst_hbm.at[pl.ds(base, K)], didx0)
    pltpu.async_copy(hs_hbm.at[sidx0], rows0, sg0)
    pltpu.async_copy(src_hbm.at[pl.ds(base + K, K)], sidx1, si1)
    pltpu.async_copy(dst_hbm.at[pl.ds(base + K, K)], didx1, sd1)
    pltpu.make_async_copy(hs_hbm.at[sidx0], rows0, sg0).wait()
    pltpu.async_copy(rows0, acc.at[didx0], ss0, add=True)

    @pl.loop(0, (CHUNKS_PER_SUB - 3) // 2)
    def _(jj):
        for i in range(2):
            j = 1 + jj * 2 + i
            _body(j, (1 + i) % 2, True)

    _body(CHUNKS_PER_SUB - 2, (CHUNKS_PER_SUB - 2) % 2, True)
    _body(CHUNKS_PER_SUB - 1, (CHUNKS_PER_SUB - 1) % 2, False)
    ob = (CHUNKS_PER_SUB - 2) % 2
    pltpu.make_async_copy(slots[ob][2], acc.at[slots[ob][1]],
                          slots[ob][6]).wait()
    lb = (CHUNKS_PER_SUB - 1) % 2
    pltpu.make_async_copy(slots[lb][2], acc.at[slots[lb][1]],
                          slots[lb][6]).wait()

    plsc.subcore_barrier()

    @pl.loop(0, NZ)
    def _(b):
        r = s * ROWS_PER_SUB + b * ZROWS
        pltpu.sync_copy(acc.at[pl.ds(r, ZROWS)], out_hbm.at[c, pl.ds(r, ZROWS)])


def _mm_body(x_ref, w_ref, o_ref):
    o_ref[...] = jnp.dot(x_ref[...], w_ref[...], preferred_element_type=_f32)


def _mm(x, w):
    return pl.pallas_call(
        _mm_body,
        grid=(GRID,),
        in_specs=[pl.BlockSpec((BLK, FEAT), lambda i: (i, 0)),
                  pl.BlockSpec((FEAT, FEAT), lambda i: (0, 0))],
        out_specs=pl.BlockSpec((BLK, FEAT), lambda i: (i, 0)),
        out_shape=jax.ShapeDtypeStruct((NP, FEAT), _f32),
    )(x, w)


def _scale_body(degp_ref, h_ref, hs_ref, dinv_ref):
    dinv = lax.rsqrt(degp_ref[0] + degp_ref[1] + 1.0)
    dinv_ref[...] = dinv
    hs_ref[...] = h_ref[...] * dinv


def _scale(degp, h):
    return pl.pallas_call(
        _scale_body,
        grid=(GRID,),
        in_specs=[pl.BlockSpec((NCORES, BLK, FEAT), lambda i: (0, i, 0)),
                  pl.BlockSpec((BLK, FEAT), lambda i: (i, 0))],
        out_specs=[pl.BlockSpec((BLK, FEAT), lambda i: (i, 0)),
                   pl.BlockSpec((BLK, FEAT), lambda i: (i, 0))],
        out_shape=[jax.ShapeDtypeStruct((NP, FEAT), _f32),
                   jax.ShapeDtypeStruct((NP, FEAT), _f32)],
    )(degp, h)


def _dense2_body(dinv_ref, accp_ref, h1_ref, b1_ref, w2_ref, h2_ref, hs2_ref):
    dinv = dinv_ref[...]
    ap = accp_ref[...]
    out1 = (ap[0] + ap[1]) * dinv + h1_ref[...] * dinv * dinv + b1_ref[...]
    h2 = jnp.dot(out1, w2_ref[...], preferred_element_type=_f32)
    h2_ref[...] = h2
    hs2_ref[...] = h2 * dinv


def _dense2(dinv, accp, h1, b1, w2):
    return pl.pallas_call(
        _dense2_body,
        grid=(GRID,),
        in_specs=[pl.BlockSpec((BLK, FEAT), lambda i: (i, 0)),
                  pl.BlockSpec((NCORES, BLK, FEAT), lambda i: (0, i, 0)),
                  pl.BlockSpec((BLK, FEAT), lambda i: (i, 0)),
                  pl.BlockSpec((1, FEAT), lambda i: (0, 0)),
                  pl.BlockSpec((FEAT, FEAT), lambda i: (0, 0))],
        out_specs=[pl.BlockSpec((BLK, FEAT), lambda i: (i, 0)),
                   pl.BlockSpec((BLK, FEAT), lambda i: (i, 0))],
        out_shape=[jax.ShapeDtypeStruct((NP, FEAT), _f32),
                   jax.ShapeDtypeStruct((NP, FEAT), _f32)],
    )(dinv, accp, h1, b1, w2)


def _final_body(dinv_ref, accp_ref, h2_ref, b2_ref, y_ref):
    dinv = dinv_ref[...]
    ap = accp_ref[...]
    out2 = (ap[0] + ap[1]) * dinv + h2_ref[...] * dinv * dinv + b2_ref[...]
    m = jnp.max(out2, axis=-1, keepdims=True)
    z = out2 - m
    y_ref[...] = z - jnp.log(jnp.sum(jnp.exp(z), axis=-1, keepdims=True))


def _final(dinv, accp, h2, b2):
    return pl.pallas_call(
        _final_body,
        grid=(GRID,),
        in_specs=[pl.BlockSpec((BLK, FEAT), lambda i: (i, 0)),
                  pl.BlockSpec((NCORES, BLK, FEAT), lambda i: (0, i, 0)),
                  pl.BlockSpec((BLK, FEAT), lambda i: (i, 0)),
                  pl.BlockSpec((1, FEAT), lambda i: (0, 0))],
        out_specs=pl.BlockSpec((BLK, FEAT), lambda i: (i, 0)),
        out_shape=jax.ShapeDtypeStruct((NP, FEAT), _f32),
    )(dinv, accp, h2, b2)


def kernel(x, edge_index, W1, b1, W2, b2):
    xp = jnp.zeros((NP, FEAT), _f32).at[:N_NODES].set(x)
    pad = jnp.full((EP - E_EDGES,), N_NODES, jnp.int32)
    src = jnp.concatenate([edge_index[0], pad])
    dst = jnp.concatenate([edge_index[1], pad])
    ones128 = jnp.ones((K, FEAT), _f32)
    zeros128 = jnp.zeros((ZROWS, FEAT), _f32)

    degp = _deg_kernel(dst, ones128, zeros128)
    h1 = _mm(xp, W1)
    hs1, dinv = _scale(degp, h1)
    acc1 = _edge_kernel(hs1, src, dst, zeros128)
    h2, hs2 = _dense2(dinv, acc1, h1, b1.reshape(1, FEAT), W2)
    acc2 = _edge_kernel(hs2, src, dst, zeros128)
    y = _final(dinv, acc2, h2, b2.reshape(1, FEAT))
    return y[:N_NODES]


# speedup vs baseline: 1.6388x; 1.0003x over previous
import functools

import jax
import jax.numpy as jnp
from jax import lax
from jax.experimental import pallas as pl
from jax.experimental.pallas import tpu as pltpu
from jax.experimental.pallas import tpu_sc as plsc

N_NODES = 10000
FEAT = 128
E_EDGES = 320000

NCORES = 2
NSUB = 16
K = 128
CHUNKS_PER_SUB = 79
CHUNKS_PER_CORE = CHUNKS_PER_SUB * NSUB
EP = CHUNKS_PER_SUB * NCORES * NSUB * K

NP = 10240
ROWS_PER_SUB = NP // NSUB
ZROWS = 128
NZ = ROWS_PER_SUB // ZROWS

BLK = 2048
GRID = NP // BLK

_mesh = plsc.VectorSubcoreMesh(core_axis_name="c", subcore_axis_name="s")
_f32 = jnp.float32


@functools.partial(
    pl.kernel,
    out_type=jax.ShapeDtypeStruct((NCORES, NP, FEAT), _f32),
    mesh=_mesh,
    scratch_types=[
        pltpu.VMEM((K,), jnp.int32),
        pltpu.VMEM((K,), jnp.int32),
        pltpu.VMEM((K, FEAT), _f32),
        pltpu.SemaphoreType.DMA,
        pltpu.SemaphoreType.DMA,
        pltpu.SemaphoreType.DMA,
        pltpu.SemaphoreType.DMA,
        pltpu.VMEM_SHARED((NP, FEAT), _f32),
    ],
)
def _deg_kernel(dst_hbm, ones_hbm, zeros_hbm, out_hbm,
                didx0, didx1, ov, si0, si1, ss0, ss1, acc):
    c = lax.axis_index("c")
    s = lax.axis_index("s")
    pltpu.sync_copy(zeros_hbm, ov.at[pl.ds(0, ZROWS)])

    @pl.loop(0, NZ)
    def _(b):
        pltpu.sync_copy(ov.at[pl.ds(0, ZROWS)],
                        acc.at[pl.ds(s * ROWS_PER_SUB + b * ZROWS, ZROWS)])

    pltpu.sync_copy(ones_hbm, ov)
    plsc.subcore_barrier()
    base = (c * CHUNKS_PER_CORE + s * CHUNKS_PER_SUB) * K
    slots = ((didx0, si0, ss0), (didx1, si1, ss1))

    def _body(j, b, prefetch):
        didx, si, ss = slots[b]
        odidx, osi, oss = slots[1 - b]
        pltpu.make_async_copy(dst_hbm.at[pl.ds(base + j * K, K)],
                              didx, si).wait()
        pltpu.async_copy(ov, acc.at[didx], ss, add=True)
        if prefetch:
            pltpu.make_async_copy(ov, acc.at[odidx], oss).wait()
            pltpu.async_copy(dst_hbm.at[pl.ds(base + (j + 1) * K, K)],
                             odidx, osi)

    pltpu.sync_copy(dst_hbm.at[pl.ds(base, K)], didx0)
    pltpu.async_copy(ov, acc.at[didx0], ss0, add=True)
    pltpu.async_copy(dst_hbm.at[pl.ds(base + K, K)], didx1, si1)

    @pl.loop(0, (CHUNKS_PER_SUB - 3) // 2)
    def _(jj):
        for i in range(2):
            _body(1 + jj * 2 + i, (1 + i) % 2, True)

    _body(CHUNKS_PER_SUB - 2, (CHUNKS_PER_SUB - 2) % 2, True)
    _body(CHUNKS_PER_SUB - 1, (CHUNKS_PER_SUB - 1) % 2, False)
    pltpu.make_async_copy(ov, acc.at[didx1], ss1).wait()
    pltpu.make_async_copy(ov, acc.at[didx0], ss0).wait()

    plsc.subcore_barrier()

    @pl.loop(0, NZ)
    def _(b):
        r = s * ROWS_PER_SUB + b * ZROWS
        pltpu.sync_copy(acc.at[pl.ds(r, ZROWS)], out_hbm.at[c, pl.ds(r, ZROWS)])


@functools.partial(
    pl.kernel,
    out_type=jax.ShapeDtypeStruct((NCORES, NP, FEAT), _f32),
    mesh=_mesh,
    scratch_types=[
        pltpu.VMEM((K,), jnp.int32),
        pltpu.VMEM((K,), jnp.int32),
        pltpu.VMEM((K,), jnp.int32),
        pltpu.VMEM((K,), jnp.int32),
        pltpu.VMEM((K, FEAT), _f32),
        pltpu.VMEM((K, FEAT), _f32),
        pltpu.SemaphoreType.DMA,
        pltpu.SemaphoreType.DMA,
        pltpu.SemaphoreType.DMA,
        pltpu.SemaphoreType.DMA,
        pltpu.SemaphoreType.DMA,
        pltpu.SemaphoreType.DMA,
        pltpu.SemaphoreType.DMA,
        pltpu.SemaphoreType.DMA,
        pltpu.VMEM_SHARED((NP, FEAT), _f32),
    ],
)
def _edge_kernel(hs_hbm, src_hbm, dst_hbm, zeros_hbm, out_hbm,
                 sidx0, sidx1, didx0, didx1, rows0, rows1,
                 si0, si1, sd0, sd1, sg0, sg1, ss0, ss1, acc):
    c = lax.axis_index("c")
    s = lax.axis_index("s")
    pltpu.sync_copy(zeros_hbm, rows0.at[pl.ds(0, ZROWS)])

    @pl.loop(0, NZ)
    def _(b):
        pltpu.sync_copy(rows0.at[pl.ds(0, ZROWS)],
                        acc.at[pl.ds(s * ROWS_PER_SUB + b * ZROWS, ZROWS)])

    plsc.subcore_barrier()
    base = (c * CHUNKS_PER_CORE + s * CHUNKS_PER_SUB) * K
    slots = ((sidx0, didx0, rows0, si0, sd0, sg0, ss0),
             (sidx1, didx1, rows1, si1, sd1, sg1, ss1))

    def _body(j, b, prefetch):
        sidx, didx, rows, si, sd, sg, ss = slots[b]
        osidx, odidx, orows, osi, osd, _, oss = slots[1 - b]
        e = base + j * K
        pltpu.make_async_copy(src_hbm.at[pl.ds(e, K)], sidx, si).wait()
        pltpu.make_async_copy(dst_hbm.at[pl.ds(e, K)], didx, sd).wait()
        pltpu.async_copy(hs_hbm.at[sidx], rows, sg)
        if prefetch:
            ne = base + (j + 1) * K
            pltpu.make_async_copy(orows, acc.at[odidx], oss).wait()
            pltpu.async_copy(src_hbm.at[pl.ds(ne, K)], osidx, osi)
            pltpu.async_copy(dst_hbm.at[pl.ds(ne, K)], odidx, osd)
        pltpu.make_async_copy(hs_hbm.at[sidx], rows, sg).wait()
        pltpu.async_copy(rows, acc.at[didx], ss, add=True)

    pltpu.sync_copy(src_hbm.at[pl.ds(base, K)], sidx0)
    pltpu.sync_copy(dst_hbm.at[pl.ds(base, K)], didx0)
    pltpu.async_copy(hs_hbm.at[sidx0], rows0, sg0)
    pltpu.async_copy(src_hbm.at[pl.ds(base + K, K)], sidx1, si1)
    pltpu.async_copy(dst_hbm.at[pl.ds(base + K, K)], didx1, sd1)
    pltpu.make_async_copy(hs_hbm.at[sidx0], rows0, sg0).wait()
    pltpu.async_copy(rows0, acc.at[didx0], ss0, add=True)

    @pl.loop(0, (CHUNKS_PER_SUB - 3) // 2)
    def _(jj):
        for i in range(2):
            j = 1 + jj * 2 + i
            _body(j, (1 + i) % 2, True)

    _body(CHUNKS_PER_SUB - 2, (CHUNKS_PER_SUB - 2) % 2, True)
    _body(CHUNKS_PER_SUB - 1, (CHUNKS_PER_SUB - 1) % 2, False)
    ob = (CHUNKS_PER_SUB - 2) % 2
    pltpu.make_async_copy(slots[ob][2], acc.at[slots[ob][1]],
                          slots[ob][6]).wait()
    lb = (CHUNKS_PER_SUB - 1) % 2
    pltpu.make_async_copy(slots[lb][2], acc.at[slots[lb][1]],
                          slots[lb][6]).wait()

    plsc.subcore_barrier()

    @pl.loop(0, NZ)
    def _(b):
        r = s * ROWS_PER_SUB + b * ZROWS
        pltpu.sync_copy(acc.at[pl.ds(r, ZROWS)], out_hbm.at[c, pl.ds(r, ZROWS)])


def _mm_body(x_ref, w_ref, o_ref):
    o_ref[...] = jnp.dot(x_ref[...], w_ref[...], preferred_element_type=_f32)


def _mm(x, w):
    return pl.pallas_call(
        _mm_body,
        grid=(GRID,),
        in_specs=[pl.BlockSpec((BLK, FEAT), lambda i: (i, 0)),
                  pl.BlockSpec((FEAT, FEAT), lambda i: (0, 0))],
        out_specs=pl.BlockSpec((BLK, FEAT), lambda i: (i, 0)),
        out_shape=jax.ShapeDtypeStruct((NP, FEAT), _f32),
    )(x, w)


def _scale_body(degp_ref, h_ref, hs_ref, dinv_ref):
    dinv = lax.rsqrt(degp_ref[0] + degp_ref[1] + 1.0)
    dinv_ref[...] = dinv
    hs_ref[...] = h_ref[...] * dinv


def _scale(degp, h):
    return pl.pallas_call(
        _scale_body,
        grid=(GRID,),
        in_specs=[pl.BlockSpec((NCORES, BLK, FEAT), lambda i: (0, i, 0)),
                  pl.BlockSpec((BLK, FEAT), lambda i: (i, 0))],
        out_specs=[pl.BlockSpec((BLK, FEAT), lambda i: (i, 0)),
                   pl.BlockSpec((BLK, FEAT), lambda i: (i, 0))],
        out_shape=[jax.ShapeDtypeStruct((NP, FEAT), _f32),
                   jax.ShapeDtypeStruct((NP, FEAT), _f32)],
    )(degp, h)


def _dense2_body(dinv_ref, accp_ref, h1_ref, b1_ref, w2_ref, h2_ref, hs2_ref):
    dinv = dinv_ref[...]
    ap = accp_ref[...]
    out1 = (ap[0] + ap[1]) * dinv + h1_ref[...] * dinv * dinv + b1_ref[...]
    h2 = jnp.dot(out1, w2_ref[...], preferred_element_type=_f32)
    h2_ref[...] = h2
    hs2_ref[...] = h2 * dinv


def _dense2(dinv, accp, h1, b1, w2):
    return pl.pallas_call(
        _dense2_body,
        grid=(GRID,),
        in_specs=[pl.BlockSpec((BLK, FEAT), lambda i: (i, 0)),
                  pl.BlockSpec((NCORES, BLK, FEAT), lambda i: (0, i, 0)),
                  pl.BlockSpec((BLK, FEAT), lambda i: (i, 0)),
                  pl.BlockSpec((1, FEAT), lambda i: (0, 0)),
                  pl.BlockSpec((FEAT, FEAT), lambda i: (0, 0))],
        out_specs=[pl.BlockSpec((BLK, FEAT), lambda i: (i, 0)),
                   pl.BlockSpec((BLK, FEAT), lambda i: (i, 0))],
        out_shape=[jax.ShapeDtypeStruct((NP, FEAT), _f32),
                   jax.ShapeDtypeStruct((NP, FEAT), _f32)],
    )(dinv, accp, h1, b1, w2)


def _final_body(dinv_ref, accp_ref, h2_ref, b2_ref, y_ref):
    dinv = dinv_ref[...]
    ap = accp_ref[...]
    out2 = (ap[0] + ap[1]) * dinv + h2_ref[...] * dinv * dinv + b2_ref[...]
    m = jnp.max(out2, axis=-1, keepdims=True)
    z = out2 - m
    y_ref[...] = z - jnp.log(jnp.sum(jnp.exp(z), axis=-1, keepdims=True))


def _final(dinv, accp, h2, b2):
    return pl.pallas_call(
        _final_body,
        grid=(GRID,),
        in_specs=[pl.BlockSpec((BLK, FEAT), lambda i: (i, 0)),
                  pl.BlockSpec((NCORES, BLK, FEAT), lambda i: (0, i, 0)),
                  pl.BlockSpec((BLK, FEAT), lambda i: (i, 0)),
                  pl.BlockSpec((1, FEAT), lambda i: (0, 0))],
        out_specs=pl.BlockSpec((BLK, FEAT), lambda i: (i, 0)),
        out_shape=jax.ShapeDtypeStruct((NP, FEAT), _f32),
    )(dinv, accp, h2, b2)


def kernel(x, edge_index, W1, b1, W2, b2):
    xp = jnp.zeros((NP, FEAT), _f32).at[:N_NODES].set(x)
    pad = jnp.full((EP - E_EDGES,), N_NODES, jnp.int32)
    src = jnp.concatenate([edge_index[0], pad])
    dst = jnp.concatenate([edge_index[1], pad])
    ones128 = jnp.ones((K, FEAT), _f32)
    zeros128 = jnp.zeros((ZROWS, FEAT), _f32)

    degp = _deg_kernel(dst, ones128, zeros128)
    h1 = _mm(xp, W1)
    hs1, dinv = _scale(degp, h1)
    acc1 = _edge_kernel(hs1, src, dst, zeros128)
    h2, hs2 = _dense2(dinv, acc1, h1, b1.reshape(1, FEAT), W2)
    acc2 = _edge_kernel(hs2, src, dst, zeros128)
    y = _final(dinv, acc2, h2, b2.reshape(1, FEAT))
    return y[:N_NODES]


# speedup vs baseline: 1.6947x; 1.0341x over previous
import functools

import jax
import jax.numpy as jnp
from jax import lax
from jax.experimental import pallas as pl
from jax.experimental.pallas import tpu as pltpu
from jax.experimental.pallas import tpu_sc as plsc

N_NODES = 10000
FEAT = 128
E_EDGES = 320000

NCORES = 2
NSUB = 16
K = 128
CHUNKS_PER_SUB = 79
CHUNKS_PER_CORE = CHUNKS_PER_SUB * NSUB
EP = CHUNKS_PER_SUB * NCORES * NSUB * K

NP = 10240
ROWS_PER_SUB = NP // NSUB
ZROWS = 128
NZ = ROWS_PER_SUB // ZROWS

BLK = 2048
GRID = NP // BLK

_mesh = plsc.VectorSubcoreMesh(core_axis_name="c", subcore_axis_name="s")
_f32 = jnp.float32


@functools.partial(
    pl.kernel,
    out_type=jax.ShapeDtypeStruct((NCORES, NP, FEAT), _f32),
    mesh=_mesh,
    scratch_types=[
        pltpu.VMEM((K,), jnp.int32),
        pltpu.VMEM((K,), jnp.int32),
        pltpu.VMEM((K, FEAT), _f32),
        pltpu.SemaphoreType.DMA,
        pltpu.SemaphoreType.DMA,
        pltpu.SemaphoreType.DMA,
        pltpu.SemaphoreType.DMA,
        pltpu.VMEM_SHARED((NP, FEAT), _f32),
    ],
)
def _deg_kernel(dst_hbm, ones_hbm, zeros_hbm, out_hbm,
                didx0, didx1, ov, si0, si1, ss0, ss1, acc):
    c = lax.axis_index("c")
    s = lax.axis_index("s")
    pltpu.sync_copy(zeros_hbm, ov.at[pl.ds(0, ZROWS)])

    @pl.loop(0, NZ)
    def _(b):
        pltpu.sync_copy(ov.at[pl.ds(0, ZROWS)],
                        acc.at[pl.ds(s * ROWS_PER_SUB + b * ZROWS, ZROWS)])

    pltpu.sync_copy(ones_hbm, ov)
    plsc.subcore_barrier()
    base = (c * CHUNKS_PER_CORE + s * CHUNKS_PER_SUB) * K
    slots = ((didx0, si0, ss0), (didx1, si1, ss1))

    def _body(j, b, prefetch):
        didx, si, ss = slots[b]
        odidx, osi, oss = slots[1 - b]
        pltpu.make_async_copy(dst_hbm.at[pl.ds(base + j * K, K)],
                              didx, si).wait()
        pltpu.async_copy(ov, acc.at[didx], ss, add=True)
        if prefetch:
            pltpu.make_async_copy(ov, acc.at[odidx], oss).wait()
            pltpu.async_copy(dst_hbm.at[pl.ds(base + (j + 1) * K, K)],
                             odidx, osi)

    pltpu.sync_copy(dst_hbm.at[pl.ds(base, K)], didx0)
    pltpu.async_copy(ov, acc.at[didx0], ss0, add=True)
    pltpu.async_copy(dst_hbm.at[pl.ds(base + K, K)], didx1, si1)

    @pl.loop(0, (CHUNKS_PER_SUB - 3) // 2)
    def _(jj):
        for i in range(2):
            _body(1 + jj * 2 + i, (1 + i) % 2, True)

    _body(CHUNKS_PER_SUB - 2, (CHUNKS_PER_SUB - 2) % 2, True)
    _body(CHUNKS_PER_SUB - 1, (CHUNKS_PER_SUB - 1) % 2, False)
    pltpu.make_async_copy(ov, acc.at[didx1], ss1).wait()
    pltpu.make_async_copy(ov, acc.at[didx0], ss0).wait()

    plsc.subcore_barrier()

    @pl.loop(0, NZ)
    def _(b):
        r = s * ROWS_PER_SUB + b * ZROWS
        pltpu.sync_copy(acc.at[pl.ds(r, ZROWS)], out_hbm.at[c, pl.ds(r, ZROWS)])


@functools.partial(
    pl.kernel,
    out_type=jax.ShapeDtypeStruct((NCORES, NP, FEAT), _f32),
    mesh=_mesh,
    scratch_types=[
        pltpu.VMEM((K,), jnp.int32),
        pltpu.VMEM((K,), jnp.int32),
        pltpu.VMEM((K,), jnp.int32),
        pltpu.VMEM((K,), jnp.int32),
        pltpu.VMEM((K, FEAT), _f32),
        pltpu.VMEM((K, FEAT), _f32),
        pltpu.SemaphoreType.DMA,
        pltpu.SemaphoreType.DMA,
        pltpu.SemaphoreType.DMA,
        pltpu.SemaphoreType.DMA,
        pltpu.SemaphoreType.DMA,
        pltpu.SemaphoreType.DMA,
        pltpu.SemaphoreType.DMA,
        pltpu.SemaphoreType.DMA,
        pltpu.VMEM_SHARED((NP, FEAT), _f32),
    ],
)
def _edge_kernel(hs_hbm, src_hbm, dst_hbm, zeros_hbm, out_hbm,
                 sidx0, sidx1, didx0, didx1, rows0, rows1,
                 si0, si1, sd0, sd1, sg0, sg1, ss0, ss1, acc):
    c = lax.axis_index("c")
    s = lax.axis_index("s")
    pltpu.sync_copy(zeros_hbm, rows0.at[pl.ds(0, ZROWS)])

    @pl.loop(0, NZ)
    def _(b):
        pltpu.sync_copy(rows0.at[pl.ds(0, ZROWS)],
                        acc.at[pl.ds(s * ROWS_PER_SUB + b * ZROWS, ZROWS)])

    plsc.subcore_barrier()
    base = (c * CHUNKS_PER_CORE + s * CHUNKS_PER_SUB) * K
    slots = ((sidx0, didx0, rows0, si0, sd0, sg0, ss0),
             (sidx1, didx1, rows1, si1, sd1, sg1, ss1))

    def _body(j, b):
        sidx, didx, rows, si, sd, sg, ss = slots[b]
        osidx, odidx, orows, osi, osd, osg, oss = slots[1 - b]
        ne = base + (j + 1) * K
        pltpu.make_async_copy(orows, acc.at[odidx], oss).wait()
        pltpu.async_copy(src_hbm.at[pl.ds(ne, K)], osidx, osi)
        pltpu.async_copy(dst_hbm.at[pl.ds(ne, K)], odidx, osd)
        pltpu.make_async_copy(src_hbm.at[pl.ds(ne, K)], osidx, osi).wait()
        pltpu.make_async_copy(dst_hbm.at[pl.ds(ne, K)], odidx, osd).wait()
        pltpu.async_copy(hs_hbm.at[osidx], orows, osg)
        pltpu.make_async_copy(hs_hbm.at[sidx], rows, sg).wait()
        pltpu.async_copy(rows, acc.at[didx], ss, add=True)

    pltpu.sync_copy(src_hbm.at[pl.ds(base, K)], sidx0)
    pltpu.sync_copy(dst_hbm.at[pl.ds(base, K)], didx0)
    pltpu.async_copy(hs_hbm.at[sidx0], rows0, sg0)
    pltpu.async_copy(src_hbm.at[pl.ds(base + K, K)], sidx1, si1)
    pltpu.async_copy(dst_hbm.at[pl.ds(base + K, K)], didx1, sd1)
    pltpu.make_async_copy(src_hbm.at[pl.ds(base + K, K)], sidx1, si1).wait()
    pltpu.make_async_copy(dst_hbm.at[pl.ds(base + K, K)], didx1, sd1).wait()
    pltpu.async_copy(hs_hbm.at[sidx1], rows1, sg1)
    pltpu.make_async_copy(hs_hbm.at[sidx0], rows0, sg0).wait()
    pltpu.async_copy(rows0, acc.at[didx0], ss0, add=True)

    @pl.loop(0, (CHUNKS_PER_SUB - 3) // 2)
    def _(jj):
        for i in range(2):
            _body(1 + jj * 2 + i, (1 + i) % 2)

    _body(CHUNKS_PER_SUB - 2, (CHUNKS_PER_SUB - 2) % 2)
    lb = (CHUNKS_PER_SUB - 1) % 2
    sidx, didx, rows, _, _, sg, ss = slots[lb]
    osidx, odidx, orows, _, _, _, oss = slots[1 - lb]
    pltpu.make_async_copy(orows, acc.at[odidx], oss).wait()
    pltpu.make_async_copy(hs_hbm.at[sidx], rows, sg).wait()
    pltpu.async_copy(rows, acc.at[didx], ss, add=True)
    pltpu.make_async_copy(rows, acc.at[didx], ss).wait()

    plsc.subcore_barrier()

    @pl.loop(0, NZ)
    def _(b):
        r = s * ROWS_PER_SUB + b * ZROWS
        pltpu.sync_copy(acc.at[pl.ds(r, ZROWS)], out_hbm.at[c, pl.ds(r, ZROWS)])


def _mm_body(x_ref, w_ref, o_ref):
    o_ref[...] = jnp.dot(x_ref[...], w_ref[...], preferred_element_type=_f32)


def _mm(x, w):
    return pl.pallas_call(
        _mm_body,
        grid=(GRID,),
        in_specs=[pl.BlockSpec((BLK, FEAT), lambda i: (i, 0)),
                  pl.BlockSpec((FEAT, FEAT), lambda i: (0, 0))],
        out_specs=pl.BlockSpec((BLK, FEAT), lambda i: (i, 0)),
        out_shape=jax.ShapeDtypeStruct((NP, FEAT), _f32),
    )(x, w)


def _scale_body(degp_ref, h_ref, hs_ref, dinv_ref):
    dinv = lax.rsqrt(degp_ref[0] + degp_ref[1] + 1.0)
    dinv_ref[...] = dinv
    hs_ref[...] = h_ref[...] * dinv


def _scale(degp, h):
    return pl.pallas_call(
        _scale_body,
        grid=(GRID,),
        in_specs=[pl.BlockSpec((NCORES, BLK, FEAT), lambda i: (0, i, 0)),
                  pl.BlockSpec((BLK, FEAT), lambda i: (i, 0))],
        out_specs=[pl.BlockSpec((BLK, FEAT), lambda i: (i, 0)),
                   pl.BlockSpec((BLK, FEAT), lambda i: (i, 0))],
        out_shape=[jax.ShapeDtypeStruct((NP, FEAT), _f32),
                   jax.ShapeDtypeStruct((NP, FEAT), _f32)],
    )(degp, h)


def _dense2_body(dinv_ref, accp_ref, h1_ref, b1_ref, w2_ref, h2_ref, hs2_ref):
    dinv = dinv_ref[...]
    ap = accp_ref[...]
    out1 = (ap[0] + ap[1]) * dinv + h1_ref[...] * dinv * dinv + b1_ref[...]
    h2 = jnp.dot(out1, w2_ref[...], preferred_element_type=_f32)
    h2_ref[...] = h2
    hs2_ref[...] = h2 * dinv


def _dense2(dinv, accp, h1, b1, w2):
    return pl.pallas_call(
        _dense2_body,
        grid=(GRID,),
        in_specs=[pl.BlockSpec((BLK, FEAT), lambda i: (i, 0)),
                  pl.BlockSpec((NCORES, BLK, FEAT), lambda i: (0, i, 0)),
                  pl.BlockSpec((BLK, FEAT), lambda i: (i, 0)),
                  pl.BlockSpec((1, FEAT), lambda i: (0, 0)),
                  pl.BlockSpec((FEAT, FEAT), lambda i: (0, 0))],
        out_specs=[pl.BlockSpec((BLK, FEAT), lambda i: (i, 0)),
                   pl.BlockSpec((BLK, FEAT), lambda i: (i, 0))],
        out_shape=[jax.ShapeDtypeStruct((NP, FEAT), _f32),
                   jax.ShapeDtypeStruct((NP, FEAT), _f32)],
    )(dinv, accp, h1, b1, w2)


def _final_body(dinv_ref, accp_ref, h2_ref, b2_ref, y_ref):
    dinv = dinv_ref[...]
    ap = accp_ref[...]
    out2 = (ap[0] + ap[1]) * dinv + h2_ref[...] * dinv * dinv + b2_ref[...]
    m = jnp.max(out2, axis=-1, keepdims=True)
    z = out2 - m
    y_ref[...] = z - jnp.log(jnp.sum(jnp.exp(z), axis=-1, keepdims=True))


def _final(dinv, accp, h2, b2):
    return pl.pallas_call(
        _final_body,
        grid=(GRID,),
        in_specs=[pl.BlockSpec((BLK, FEAT), lambda i: (i, 0)),
                  pl.BlockSpec((NCORES, BLK, FEAT), lambda i: (0, i, 0)),
                  pl.BlockSpec((BLK, FEAT), lambda i: (i, 0)),
                  pl.BlockSpec((1, FEAT), lambda i: (0, 0))],
        out_specs=pl.BlockSpec((BLK, FEAT), lambda i: (i, 0)),
        out_shape=jax.ShapeDtypeStruct((NP, FEAT), _f32),
    )(dinv, accp, h2, b2)


def kernel(x, edge_index, W1, b1, W2, b2):
    xp = jnp.zeros((NP, FEAT), _f32).at[:N_NODES].set(x)
    pad = jnp.full((EP - E_EDGES,), N_NODES, jnp.int32)
    src = jnp.concatenate([edge_index[0], pad])
    dst = jnp.concatenate([edge_index[1], pad])
    ones128 = jnp.ones((K, FEAT), _f32)
    zeros128 = jnp.zeros((ZROWS, FEAT), _f32)

    degp = _deg_kernel(dst, ones128, zeros128)
    h1 = _mm(xp, W1)
    hs1, dinv = _scale(degp, h1)
    acc1 = _edge_kernel(hs1, src, dst, zeros128)
    h2, hs2 = _dense2(dinv, acc1, h1, b1.reshape(1, FEAT), W2)
    acc2 = _edge_kernel(hs2, src, dst, zeros128)
    y = _final(dinv, acc2, h2, b2.reshape(1, FEAT))
    return y[:N_NODES]
